# initial kernel scaffold (unmeasured)
import jax
import jax.numpy as jnp
from jax import lax
from jax.experimental import pallas as pl
from jax.experimental.pallas import tpu as pltpu

N_DEV = 4
M, K_SHARD, N = 4096, 1024, 8192
CHUNK = M // N_DEV


def _gemm_body(x_ref, w_ref, o_ref):
    o_ref[...] = jnp.dot(x_ref[...], w_ref[...],
                         preferred_element_type=jnp.float32)


def _partial_gemm(x, w):
    bm, bn = 1024, 2048
    return pl.pallas_call(
        _gemm_body,
        grid=(M // bm, N // bn),
        in_specs=[
            pl.BlockSpec((bm, K_SHARD), lambda m, n: (m, 0)),
            pl.BlockSpec((K_SHARD, bn), lambda m, n: (0, n)),
        ],
        out_specs=pl.BlockSpec((bm, bn), lambda m, n: (m, n)),
        out_shape=jax.ShapeDtypeStruct((M, N), jnp.float32),
        compiler_params=pltpu.CompilerParams(
            dimension_semantics=("parallel", "parallel"),
        ),
    )(x, w)


def _ar_body(partial_ref, out_ref,
             acc, rcv, q0, q1, amax_all,
             send_sem, recv_sem, copy_sem,
             amax_send, amax_recv, credit_sem):
    my = lax.axis_index("i")
    left = (my + N_DEV - 1) % N_DEV
    right = (my + 1) % N_DEV

    barrier = pltpu.get_barrier_semaphore()
    for nbr in (left, right):
        pl.semaphore_signal(barrier, inc=1, device_id=(nbr,),
                            device_id_type=pl.DeviceIdType.MESH)
    pl.semaphore_wait(barrier, 2)

    def nbarrier():
        for nbr in (left, right):
            pl.semaphore_signal(credit_sem, inc=1, device_id=(nbr,),
                                device_id_type=pl.DeviceIdType.MESH)
        pl.semaphore_wait(credit_sem, 2)

    def load_chunk(c, dst):
        cp = pltpu.make_async_copy(
            partial_ref.at[pl.ds(c * CHUNK, CHUNK), :], dst, copy_sem)
        cp.start()
        cp.wait()

    load_chunk(my, acc)
    for s in range(N_DEV - 1):
        rdma = pltpu.make_async_remote_copy(
            src_ref=acc, dst_ref=rcv,
            send_sem=send_sem, recv_sem=recv_sem,
            device_id=(right,), device_id_type=pl.DeviceIdType.MESH)
        rdma.start()
        rdma.wait()
        load_chunk((my - s - 1) % N_DEV, acc)
        acc[...] = acc[...] + rcv[...]
        nbarrier()


    amax = jnp.max(jnp.abs(acc[...]))
    amax_all[pl.ds(my, 1), :] = jnp.full((1, 128), amax, jnp.float32)
    for d in range(1, N_DEV):
        rd = pltpu.make_async_remote_copy(
            src_ref=amax_all.at[pl.ds(my, 1), :],
            dst_ref=amax_all.at[pl.ds(my, 1), :],
            send_sem=amax_send.at[d], recv_sem=amax_recv.at[d],
            device_id=((my + d) % N_DEV,),
            device_id_type=pl.DeviceIdType.MESH)
        rd.start()
    for d in range(1, N_DEV):
        wr = pltpu.make_async_remote_copy(
            src_ref=amax_all.at[pl.ds(my, 1), :],
            dst_ref=amax_all.at[pl.ds((my - d) % N_DEV, 1), :],
            send_sem=amax_send.at[d], recv_sem=amax_recv.at[d],
            device_id=(left,), device_id_type=pl.DeviceIdType.MESH)
        wr.wait_recv()
        wr.wait_send()

    gmax = jnp.max(amax_all[...])
    inv_scale = 127.0 / gmax
    scale = gmax / 127.0

    q0[...] = jnp.clip(jnp.round(acc[...] * inv_scale),
                       -127.0, 127.0).astype(jnp.int8)
    rcv[...] = q0[...].astype(jnp.float32) * scale
    own = (my + 1) % N_DEV
    st = pltpu.make_async_copy(
        rcv, out_ref.at[pl.ds(own * CHUNK, CHUNK), :], copy_sem)
    st.start()
    st.wait()

    qbufs = (q0, q1)
    for s in range(N_DEV - 1):
        rdma = pltpu.make_async_remote_copy(
            src_ref=qbufs[s % 2], dst_ref=qbufs[(s + 1) % 2],
            send_sem=send_sem, recv_sem=recv_sem,
            device_id=(right,), device_id_type=pl.DeviceIdType.MESH)
        rdma.start()
        rdma.wait()
        c = (my - s) % N_DEV
        rcv[...] = qbufs[(s + 1) % 2][...].astype(jnp.float32) * scale
        st = pltpu.make_async_copy(
            rcv, out_ref.at[pl.ds(c * CHUNK, CHUNK), :], copy_sem)
        st.start()
        st.wait()
        nbarrier()


def _all_reduce_quant(partial):
    return pl.pallas_call(
        _ar_body,
        in_specs=[pl.BlockSpec(memory_space=pl.ANY)],
        out_specs=pl.BlockSpec(memory_space=pl.ANY),
        out_shape=jax.ShapeDtypeStruct((M, N), jnp.float32),
        scratch_shapes=[
            pltpu.VMEM((CHUNK, N), jnp.float32),
            pltpu.VMEM((CHUNK, N), jnp.float32),
            pltpu.VMEM((CHUNK, N), jnp.int8),
            pltpu.VMEM((CHUNK, N), jnp.int8),
            pltpu.VMEM((N_DEV, 128), jnp.float32),
            pltpu.SemaphoreType.DMA,
            pltpu.SemaphoreType.DMA,
            pltpu.SemaphoreType.DMA,
            pltpu.SemaphoreType.DMA((N_DEV,)),
            pltpu.SemaphoreType.DMA((N_DEV,)),
            pltpu.SemaphoreType.REGULAR,
        ],
        compiler_params=pltpu.CompilerParams(collective_id=0),
    )(partial)


def kernel(x, w_mat):
    partial = _partial_gemm(x, w_mat)
    return _all_reduce_quant(partial)


# baseline (device time: 1761200 ns/iter reference)
import jax
import jax.numpy as jnp
from jax import lax
from jax.experimental import pallas as pl
from jax.experimental.pallas import tpu as pltpu

N_DEV = 4
M, K_SHARD, N = 4096, 1024, 8192
CHUNK = M // N_DEV
SUB = 256
N_SUB = CHUNK // SUB


def _gemm_body(x_ref, w_ref, o_ref):
    o_ref[...] = jnp.dot(x_ref[...], w_ref[...],
                         preferred_element_type=jnp.float32)


def _partial_gemm(x, w):
    bm, bn = 1024, 2048
    return pl.pallas_call(
        _gemm_body,
        grid=(M // bm, N // bn),
        in_specs=[
            pl.BlockSpec((bm, K_SHARD), lambda m, n: (m, 0)),
            pl.BlockSpec((K_SHARD, bn), lambda m, n: (0, n)),
        ],
        out_specs=pl.BlockSpec((bm, bn), lambda m, n: (m, n)),
        out_shape=jax.ShapeDtypeStruct((M, N), jnp.float32),
        compiler_params=pltpu.CompilerParams(
            dimension_semantics=("parallel", "parallel"),
            vmem_limit_bytes=60 * 1024 * 1024,
        ),
    )(x, w)


def _ar_body(partial_ref, out_ref, acc_hbm, recv_hbm,
             va, vb, q0, q1, amax_all,
             send_sem, recv_sem, cpa_sem, cpb_sem,
             amax_send, amax_recv, credit_sem):
    my = lax.axis_index("i")
    left = (my + N_DEV - 1) % N_DEV
    right = (my + 1) % N_DEV

    barrier = pltpu.get_barrier_semaphore()
    for nbr in (left, right):
        pl.semaphore_signal(barrier, inc=1, device_id=(nbr,),
                            device_id_type=pl.DeviceIdType.MESH)
    pl.semaphore_wait(barrier, 2)

    def nbarrier():
        for nbr in (left, right):
            pl.semaphore_signal(credit_sem, inc=1, device_id=(nbr,),
                                device_id_type=pl.DeviceIdType.MESH)
        pl.semaphore_wait(credit_sem, 2)

    def copy(src, dst, sem):
        cp = pltpu.make_async_copy(src, dst, sem)
        cp.start()
        return cp

    amax = jnp.float32(0.0)

    for s in range(N_DEV - 1):
        src = partial_ref.at[pl.ds(my * CHUNK, CHUNK), :] if s == 0 else acc_hbm
        rdma = pltpu.make_async_remote_copy(
            src_ref=src, dst_ref=recv_hbm,
            send_sem=send_sem, recv_sem=recv_sem,
            device_id=(right,), device_id_type=pl.DeviceIdType.MESH)
        rdma.start()
        rdma.wait()
        c = (my - s - 1) % N_DEV
        for k in range(N_SUB):
            rows = pl.ds(k * SUB, SUB)
            prow = pl.ds(c * CHUNK + k * SUB, SUB)
            ca = copy(recv_hbm.at[rows, :], va, cpa_sem)
            cb = copy(partial_ref.at[prow, :], vb, cpb_sem)
            ca.wait()
            cb.wait()
            vb[...] = vb[...] + va[...]
            if s == N_DEV - 2:
                amax = jnp.maximum(amax, jnp.max(jnp.abs(vb[...])))
            copy(vb, acc_hbm.at[rows, :], cpa_sem).wait()
        nbarrier()


    amax_all[pl.ds(my, 1), :] = jnp.full((1, 128), amax, jnp.float32)
    for d in range(1, N_DEV):
        rd = pltpu.make_async_remote_copy(
            src_ref=amax_all.at[pl.ds(my, 1), :],
            dst_ref=amax_all.at[pl.ds(my, 1), :],
            send_sem=amax_send.at[d], recv_sem=amax_recv.at[d],
            device_id=((my + d) % N_DEV,),
            device_id_type=pl.DeviceIdType.MESH)
        rd.start()
    for d in range(1, N_DEV):
        wr = pltpu.make_async_remote_copy(
            src_ref=amax_all.at[pl.ds(my, 1), :],
            dst_ref=amax_all.at[pl.ds((my - d) % N_DEV, 1), :],
            send_sem=amax_send.at[d], recv_sem=amax_recv.at[d],
            device_id=(left,), device_id_type=pl.DeviceIdType.MESH)
        wr.wait_recv()
        wr.wait_send()

    gmax = jnp.max(amax_all[...])
    inv_scale = 127.0 / gmax
    scale = gmax / 127.0

    own = (my + 1) % N_DEV
    for k in range(N_SUB):
        rows = pl.ds(k * SUB, SUB)
        copy(acc_hbm.at[rows, :], va, cpa_sem).wait()
        qv = jnp.clip(jnp.round(va[...] * inv_scale), -127.0, 127.0)
        q0[rows, :] = qv.astype(jnp.int8)
        vb[...] = qv * scale
        copy(vb, out_ref.at[pl.ds(own * CHUNK + k * SUB, SUB), :],
             cpb_sem).wait()

    qbufs = (q0, q1)
    for s in range(N_DEV - 1):
        rdma = pltpu.make_async_remote_copy(
            src_ref=qbufs[s % 2], dst_ref=qbufs[(s + 1) % 2],
            send_sem=send_sem, recv_sem=recv_sem,
            device_id=(right,), device_id_type=pl.DeviceIdType.MESH)
        rdma.start()
        rdma.wait()
        c = (my - s) % N_DEV
        for k in range(N_SUB):
            rows = pl.ds(k * SUB, SUB)
            va[...] = qbufs[(s + 1) % 2][rows, :].astype(jnp.float32) * scale
            copy(va, out_ref.at[pl.ds(c * CHUNK + k * SUB, SUB), :],
                 cpa_sem).wait()
        nbarrier()


def _all_reduce_quant(partial):
    out, _, _ = pl.pallas_call(
        _ar_body,
        in_specs=[pl.BlockSpec(memory_space=pl.ANY)],
        out_specs=[pl.BlockSpec(memory_space=pl.ANY)] * 3,
        out_shape=[
            jax.ShapeDtypeStruct((M, N), jnp.float32),
            jax.ShapeDtypeStruct((CHUNK, N), jnp.float32),
            jax.ShapeDtypeStruct((CHUNK, N), jnp.float32),
        ],
        scratch_shapes=[
            pltpu.VMEM((SUB, N), jnp.float32),
            pltpu.VMEM((SUB, N), jnp.float32),
            pltpu.VMEM((CHUNK, N), jnp.int8),
            pltpu.VMEM((CHUNK, N), jnp.int8),
            pltpu.VMEM((N_DEV, 128), jnp.float32),
            pltpu.SemaphoreType.DMA,
            pltpu.SemaphoreType.DMA,
            pltpu.SemaphoreType.DMA,
            pltpu.SemaphoreType.DMA,
            pltpu.SemaphoreType.DMA((N_DEV,)),
            pltpu.SemaphoreType.DMA((N_DEV,)),
            pltpu.SemaphoreType.REGULAR,
        ],
        compiler_params=pltpu.CompilerParams(
            collective_id=0,
            vmem_limit_bytes=60 * 1024 * 1024,
        ),
    )(partial)
    return out


def kernel(x, w_mat):
    partial = _partial_gemm(x, w_mat)
    return _all_reduce_quant(partial)


# device time: 1660436 ns/iter; 1.0607x vs baseline; 1.0607x over previous
import jax
import jax.numpy as jnp
from jax import lax
from jax.experimental import pallas as pl
from jax.experimental.pallas import tpu as pltpu

N_DEV = 4
M, K_SHARD, N = 4096, 1024, 8192
CHUNK = M // N_DEV
BN = 2048
NT = N // BN


def _start_copy(src, dst, sem):
    cp = pltpu.make_async_copy(src, dst, sem)
    cp.start()
    return cp


def _ar_body(x_ref, w_ref, out_ref, acc_hbm, pacc_hbm, recv_hbm,
             xc, wt, ot, q0, q1, amax_all,
             xc_sem, wt_sem, ot_sem, send_sem, recv_sem,
             amax_send, amax_recv, credit_sem):
    my = lax.axis_index("i")
    left = (my + N_DEV - 1) % N_DEV
    right = (my + 1) % N_DEV

    def nbarrier():
        for nbr in (left, right):
            pl.semaphore_signal(credit_sem, inc=1, device_id=(nbr,),
                                device_id_type=pl.DeviceIdType.MESH)
        pl.semaphore_wait(credit_sem, 2)

    def w_tile(n):
        return w_ref.at[:, pl.ds(n * BN, BN)]

    def gemm_chunk(c, dst):
        cpx = _start_copy(x_ref.at[pl.ds(c * CHUNK, CHUNK), :], xc, xc_sem)
        _start_copy(w_tile(0), wt.at[0], wt_sem.at[0])
        cpx.wait()

        def body(n, carry):
            b = n % 2
            pltpu.make_async_copy(w_tile(n), wt.at[b], wt_sem.at[b]).wait()

            @pl.when(n + 1 < NT)
            def _():
                pltpu.make_async_copy(w_tile(n + 1), wt.at[1 - b],
                                      wt_sem.at[1 - b]).start()

            @pl.when(n >= 2)
            def _():
                pltpu.make_async_copy(
                    ot.at[b], dst.at[:, pl.ds((n - 2) * BN, BN)],
                    ot_sem.at[b]).wait()

            ot[b, :, :] = jnp.dot(xc[...], wt[b, :, :],
                                  preferred_element_type=jnp.float32)
            pltpu.make_async_copy(ot.at[b], dst.at[:, pl.ds(n * BN, BN)],
                                  ot_sem.at[b]).start()
            return carry

        lax.fori_loop(0, NT, body, 0)
        pltpu.make_async_copy(ot.at[0], dst.at[:, pl.ds((NT - 2) * BN, BN)],
                              ot_sem.at[0]).wait()
        pltpu.make_async_copy(ot.at[1], dst.at[:, pl.ds((NT - 1) * BN, BN)],
                              ot_sem.at[1]).wait()

    def add_pass():
        def body(n, am):
            cols = pl.ds(n * BN, BN)
            c0 = _start_copy(recv_hbm.at[:, cols], wt.at[0], wt_sem.at[0])
            c1 = _start_copy(pacc_hbm.at[:, cols], wt.at[1], wt_sem.at[1])
            c0.wait()
            c1.wait()
            wt[1, :, :] = wt[0, :, :] + wt[1, :, :]
            am = jnp.maximum(am, jnp.max(jnp.abs(wt[1, :, :])))
            _start_copy(wt.at[1], acc_hbm.at[:, cols], ot_sem.at[0]).wait()
            return am

        return lax.fori_loop(0, NT, body, jnp.float32(0.0))

    gemm_chunk(my, acc_hbm)

    barrier = pltpu.get_barrier_semaphore()
    for nbr in (left, right):
        pl.semaphore_signal(barrier, inc=1, device_id=(nbr,),
                            device_id_type=pl.DeviceIdType.MESH)
    pl.semaphore_wait(barrier, 2)

    def rs_body(s, amax):
        rdma = pltpu.make_async_remote_copy(
            src_ref=acc_hbm, dst_ref=recv_hbm,
            send_sem=send_sem, recv_sem=recv_sem,
            device_id=(right,), device_id_type=pl.DeviceIdType.MESH)
        rdma.start()
        gemm_chunk((my - s - 1) % N_DEV, pacc_hbm)
        rdma.wait()
        am = add_pass()
        nbarrier()
        return jnp.where(s == N_DEV - 2, am, amax)

    amax = lax.fori_loop(0, N_DEV - 1, rs_body, jnp.float32(0.0))


    amax_all[pl.ds(my, 1), :] = jnp.full((1, 128), amax, jnp.float32)
    for d in range(1, N_DEV):
        rd = pltpu.make_async_remote_copy(
            src_ref=amax_all.at[pl.ds(my, 1), :],
            dst_ref=amax_all.at[pl.ds(my, 1), :],
            send_sem=amax_send.at[d], recv_sem=amax_recv.at[d],
            device_id=((my + d) % N_DEV,),
            device_id_type=pl.DeviceIdType.MESH)
        rd.start()
    for d in range(1, N_DEV):
        wr = pltpu.make_async_remote_copy(
            src_ref=amax_all.at[pl.ds(my, 1), :],
            dst_ref=amax_all.at[pl.ds((my - d) % N_DEV, 1), :],
            send_sem=amax_send.at[d], recv_sem=amax_recv.at[d],
            device_id=(left,), device_id_type=pl.DeviceIdType.MESH)
        wr.wait_recv()
        wr.wait_send()

    gmax = jnp.max(amax_all[...])
    inv_scale = 127.0 / gmax
    scale = gmax / 127.0

    own = (my + 1) % N_DEV

    def quant_body(n, carry):
        cols = pl.ds(n * BN, BN)
        _start_copy(acc_hbm.at[:, cols], wt.at[0], wt_sem.at[0]).wait()
        q0[:, cols] = jnp.clip(jnp.round(wt[0, :, :] * inv_scale),
                               -127.0, 127.0).astype(jnp.int8)
        ot[0, :, :] = q0[:, cols].astype(jnp.float32) * scale
        _start_copy(ot.at[0], out_ref.at[pl.ds(own * CHUNK, CHUNK), cols],
                    ot_sem.at[0]).wait()
        return carry

    lax.fori_loop(0, NT, quant_body, 0)

    def dequant_store(qbuf, c):
        def body(n, carry):
            cols = pl.ds(n * BN, BN)
            ot[0, :, :] = qbuf[:, cols].astype(jnp.float32) * scale
            _start_copy(ot.at[0], out_ref.at[pl.ds(c * CHUNK, CHUNK), cols],
                        ot_sem.at[0]).wait()
            return carry

        lax.fori_loop(0, NT, body, 0)

    qbufs = (q0, q1)
    for s in range(N_DEV - 1):
        rdma = pltpu.make_async_remote_copy(
            src_ref=qbufs[s % 2], dst_ref=qbufs[(s + 1) % 2],
            send_sem=send_sem, recv_sem=recv_sem,
            device_id=(right,), device_id_type=pl.DeviceIdType.MESH)
        rdma.start()
        if s > 0:
            dequant_store(qbufs[s % 2], (my - s + 1) % N_DEV)
        rdma.wait()
        nbarrier()
    dequant_store(qbufs[(N_DEV - 1) % 2], (my - N_DEV + 2) % N_DEV)


def kernel(x, w_mat):
    out, _, _, _ = pl.pallas_call(
        _ar_body,
        in_specs=[pl.BlockSpec(memory_space=pl.ANY)] * 2,
        out_specs=[pl.BlockSpec(memory_space=pl.ANY)] * 4,
        out_shape=[
            jax.ShapeDtypeStruct((M, N), jnp.float32),
            jax.ShapeDtypeStruct((CHUNK, N), jnp.float32),
            jax.ShapeDtypeStruct((CHUNK, N), jnp.float32),
            jax.ShapeDtypeStruct((CHUNK, N), jnp.float32),
        ],
        scratch_shapes=[
            pltpu.VMEM((CHUNK, K_SHARD), jnp.float32),
            pltpu.VMEM((2, K_SHARD, BN), jnp.float32),
            pltpu.VMEM((2, CHUNK, BN), jnp.float32),
            pltpu.VMEM((CHUNK, N), jnp.int8),
            pltpu.VMEM((CHUNK, N), jnp.int8),
            pltpu.VMEM((N_DEV, 128), jnp.float32),
            pltpu.SemaphoreType.DMA,
            pltpu.SemaphoreType.DMA((2,)),
            pltpu.SemaphoreType.DMA((2,)),
            pltpu.SemaphoreType.DMA,
            pltpu.SemaphoreType.DMA,
            pltpu.SemaphoreType.DMA((N_DEV,)),
            pltpu.SemaphoreType.DMA((N_DEV,)),
            pltpu.SemaphoreType.REGULAR,
        ],
        compiler_params=pltpu.CompilerParams(
            collective_id=0,
            vmem_limit_bytes=60 * 1024 * 1024,
        ),
    )(x, w_mat)
    return out


# device time: 1660303 ns/iter; 1.0608x vs baseline; 1.0001x over previous
import jax
import jax.numpy as jnp
from jax import lax
from jax.experimental import pallas as pl
from jax.experimental.pallas import tpu as pltpu

N_DEV = 4
M, K_SHARD, N = 4096, 1024, 8192
CHUNK = M // N_DEV
BN = 2048
NT = N // BN


def _start_copy(src, dst, sem):
    cp = pltpu.make_async_copy(src, dst, sem)
    cp.start()
    return cp


def _ar_body(x_ref, w_ref, out_ref, acc_hbm, pacc_hbm, recv_hbm,
             xc, wt, ot, q0, q1, amax_all,
             xc_sem, wt_sem, ot_sem, send_sem, recv_sem,
             amax_send, amax_recv, credit_sem):
    my = lax.axis_index("i")
    left = (my + N_DEV - 1) % N_DEV
    right = (my + 1) % N_DEV

    def nbarrier():
        for nbr in (left, right):
            pl.semaphore_signal(credit_sem, inc=1, device_id=(nbr,),
                                device_id_type=pl.DeviceIdType.MESH)
        pl.semaphore_wait(credit_sem, 2)

    def w_tile(n):
        return w_ref.at[:, pl.ds(n * BN, BN)]

    def gemm_chunk(c, dst):
        cpx = _start_copy(x_ref.at[pl.ds(c * CHUNK, CHUNK), :], xc, xc_sem)
        _start_copy(w_tile(0), wt.at[0], wt_sem.at[0])
        cpx.wait()

        def body(i, carry):
            n0 = 2 * i
            n1 = n0 + 1
            pltpu.make_async_copy(w_tile(n0), wt.at[0], wt_sem.at[0]).wait()
            pltpu.make_async_copy(w_tile(n1), wt.at[1], wt_sem.at[1]).start()

            @pl.when(i >= 1)
            def _():
                pltpu.make_async_copy(
                    ot.at[0], dst.at[:, pl.ds((n0 - 2) * BN, BN)],
                    ot_sem.at[0]).wait()

            ot[0, :, :] = jnp.dot(xc[...], wt[0, :, :],
                                  preferred_element_type=jnp.float32)
            pltpu.make_async_copy(ot.at[0], dst.at[:, pl.ds(n0 * BN, BN)],
                                  ot_sem.at[0]).start()

            pltpu.make_async_copy(w_tile(n1), wt.at[1], wt_sem.at[1]).wait()

            @pl.when(i + 1 < NT // 2)
            def _():
                pltpu.make_async_copy(w_tile(n0 + 2), wt.at[0],
                                      wt_sem.at[0]).start()

            @pl.when(i >= 1)
            def _():
                pltpu.make_async_copy(
                    ot.at[1], dst.at[:, pl.ds((n1 - 2) * BN, BN)],
                    ot_sem.at[1]).wait()

            ot[1, :, :] = jnp.dot(xc[...], wt[1, :, :],
                                  preferred_element_type=jnp.float32)
            pltpu.make_async_copy(ot.at[1], dst.at[:, pl.ds(n1 * BN, BN)],
                                  ot_sem.at[1]).start()
            return carry

        lax.fori_loop(0, NT // 2, body, 0)
        pltpu.make_async_copy(ot.at[0], dst.at[:, pl.ds((NT - 2) * BN, BN)],
                              ot_sem.at[0]).wait()
        pltpu.make_async_copy(ot.at[1], dst.at[:, pl.ds((NT - 1) * BN, BN)],
                              ot_sem.at[1]).wait()

    def add_pass():
        def body(n, am):
            cols = pl.ds(n * BN, BN)
            c0 = _start_copy(recv_hbm.at[:, cols], wt.at[0], wt_sem.at[0])
            c1 = _start_copy(pacc_hbm.at[:, cols], wt.at[1], wt_sem.at[1])
            c0.wait()
            c1.wait()
            wt[1, :, :] = wt[0, :, :] + wt[1, :, :]
            am = jnp.maximum(am, jnp.max(jnp.abs(wt[1, :, :])))
            _start_copy(wt.at[1], acc_hbm.at[:, cols], ot_sem.at[0]).wait()
            return am

        return lax.fori_loop(0, NT, body, jnp.float32(0.0))

    gemm_chunk(my, acc_hbm)

    barrier = pltpu.get_barrier_semaphore()
    for nbr in (left, right):
        pl.semaphore_signal(barrier, inc=1, device_id=(nbr,),
                            device_id_type=pl.DeviceIdType.MESH)
    pl.semaphore_wait(barrier, 2)

    def rs_body(s, amax):
        rdma = pltpu.make_async_remote_copy(
            src_ref=acc_hbm, dst_ref=recv_hbm,
            send_sem=send_sem, recv_sem=recv_sem,
            device_id=(right,), device_id_type=pl.DeviceIdType.MESH)
        rdma.start()
        gemm_chunk((my - s - 1) % N_DEV, pacc_hbm)
        rdma.wait()
        am = add_pass()
        nbarrier()
        return jnp.where(s == N_DEV - 2, am, amax)

    amax = lax.fori_loop(0, N_DEV - 1, rs_body, jnp.float32(0.0))


    amax_all[pl.ds(my, 1), :] = jnp.full((1, 128), amax, jnp.float32)
    for d in range(1, N_DEV):
        rd = pltpu.make_async_remote_copy(
            src_ref=amax_all.at[pl.ds(my, 1), :],
            dst_ref=amax_all.at[pl.ds(my, 1), :],
            send_sem=amax_send.at[d], recv_sem=amax_recv.at[d],
            device_id=((my + d) % N_DEV,),
            device_id_type=pl.DeviceIdType.MESH)
        rd.start()
    for d in range(1, N_DEV):
        wr = pltpu.make_async_remote_copy(
            src_ref=amax_all.at[pl.ds(my, 1), :],
            dst_ref=amax_all.at[pl.ds((my - d) % N_DEV, 1), :],
            send_sem=amax_send.at[d], recv_sem=amax_recv.at[d],
            device_id=(left,), device_id_type=pl.DeviceIdType.MESH)
        wr.wait_recv()
        wr.wait_send()

    gmax = jnp.max(amax_all[...])
    inv_scale = 127.0 / gmax
    scale = gmax / 127.0

    own = (my + 1) % N_DEV

    def quant_body(n, carry):
        cols = pl.ds(n * BN, BN)
        _start_copy(acc_hbm.at[:, cols], wt.at[0], wt_sem.at[0]).wait()
        q0[:, cols] = jnp.clip(jnp.round(wt[0, :, :] * inv_scale),
                               -127.0, 127.0).astype(jnp.int8)
        ot[0, :, :] = q0[:, cols].astype(jnp.float32) * scale
        _start_copy(ot.at[0], out_ref.at[pl.ds(own * CHUNK, CHUNK), cols],
                    ot_sem.at[0]).wait()
        return carry

    lax.fori_loop(0, NT, quant_body, 0)

    def dequant_store(qbuf, c):
        def body(n, carry):
            cols = pl.ds(n * BN, BN)
            ot[0, :, :] = qbuf[:, cols].astype(jnp.float32) * scale
            _start_copy(ot.at[0], out_ref.at[pl.ds(c * CHUNK, CHUNK), cols],
                        ot_sem.at[0]).wait()
            return carry

        lax.fori_loop(0, NT, body, 0)

    qbufs = (q0, q1)
    for s in range(N_DEV - 1):
        rdma = pltpu.make_async_remote_copy(
            src_ref=qbufs[s % 2], dst_ref=qbufs[(s + 1) % 2],
            send_sem=send_sem, recv_sem=recv_sem,
            device_id=(right,), device_id_type=pl.DeviceIdType.MESH)
        rdma.start()
        if s > 0:
            dequant_store(qbufs[s % 2], (my - s + 1) % N_DEV)
        rdma.wait()
        nbarrier()
    dequant_store(qbufs[(N_DEV - 1) % 2], (my - N_DEV + 2) % N_DEV)


def kernel(x, w_mat):
    out, _, _, _ = pl.pallas_call(
        _ar_body,
        in_specs=[pl.BlockSpec(memory_space=pl.ANY)] * 2,
        out_specs=[pl.BlockSpec(memory_space=pl.ANY)] * 4,
        out_shape=[
            jax.ShapeDtypeStruct((M, N), jnp.float32),
            jax.ShapeDtypeStruct((CHUNK, N), jnp.float32),
            jax.ShapeDtypeStruct((CHUNK, N), jnp.float32),
            jax.ShapeDtypeStruct((CHUNK, N), jnp.float32),
        ],
        scratch_shapes=[
            pltpu.VMEM((CHUNK, K_SHARD), jnp.float32),
            pltpu.VMEM((2, K_SHARD, BN), jnp.float32),
            pltpu.VMEM((2, CHUNK, BN), jnp.float32),
            pltpu.VMEM((CHUNK, N), jnp.int8),
            pltpu.VMEM((CHUNK, N), jnp.int8),
            pltpu.VMEM((N_DEV, 128), jnp.float32),
            pltpu.SemaphoreType.DMA,
            pltpu.SemaphoreType.DMA((2,)),
            pltpu.SemaphoreType.DMA((2,)),
            pltpu.SemaphoreType.DMA,
            pltpu.SemaphoreType.DMA,
            pltpu.SemaphoreType.DMA((N_DEV,)),
            pltpu.SemaphoreType.DMA((N_DEV,)),
            pltpu.SemaphoreType.REGULAR,
        ],
        compiler_params=pltpu.CompilerParams(
            collective_id=0,
            vmem_limit_bytes=60 * 1024 * 1024,
        ),
    )(x, w_mat)
    return out


# device time: 977754 ns/iter; 1.8013x vs baseline; 1.6981x over previous
import jax
import jax.numpy as jnp
from jax import lax
from jax.experimental import pallas as pl
from jax.experimental.pallas import tpu as pltpu

N_DEV = 4
M, K_SHARD, N = 4096, 1024, 8192
CHUNK = M // N_DEV
NH = N // 2
BN = 2048
NTH = NH // BN


def _start_copy(src, dst, sem):
    cp = pltpu.make_async_copy(src, dst, sem)
    cp.start()
    return cp


def _ar_body(x_ref, w_ref, out_ref,
             acc_a, acc_b, pacc_a, pacc_b, recv_a, recv_b,
             xc, wt, ot, qa, qb, amax_all,
             xc_sem, wt_sem, ot_sem, st_sem,
             send_a, recv_a_sem, send_b, recv_b_sem,
             amax_send, amax_recv, credit_sem):
    my = lax.axis_index("i")
    left = (my + N_DEV - 1) % N_DEV
    right = (my + 1) % N_DEV

    def nbarrier():
        for nbr in (left, right):
            pl.semaphore_signal(credit_sem, inc=1, device_id=(nbr,),
                                device_id_type=pl.DeviceIdType.MESH)
        pl.semaphore_wait(credit_sem, 2)

    def gemm_half(c, off, dst, xslot):
        cpx = _start_copy(x_ref.at[pl.ds(c * CHUNK, CHUNK), :],
                          xc.at[xslot], xc_sem.at[xslot])
        w0 = _start_copy(w_ref.at[:, pl.ds(off, BN)], wt.at[0],
                         wt_sem.at[0])
        w1 = _start_copy(w_ref.at[:, pl.ds(off + BN, BN)], wt.at[1],
                         wt_sem.at[1])
        cpx.wait()
        w0.wait()
        ot[0, :, :] = jnp.dot(xc[xslot, :, :], wt[0, :, :],
                              preferred_element_type=jnp.float32)
        s0 = _start_copy(ot.at[0], dst.at[:, pl.ds(0, BN)], ot_sem.at[0])
        w1.wait()
        ot[1, :, :] = jnp.dot(xc[xslot, :, :], wt[1, :, :],
                              preferred_element_type=jnp.float32)
        s1 = _start_copy(ot.at[1], dst.at[:, pl.ds(BN, BN)], ot_sem.at[1])
        s0.wait()
        s1.wait()

    def add_pass():
        am = jnp.float32(0.0)
        for n in range(NTH):
            cols = pl.ds(n * BN, BN)
            la0 = _start_copy(recv_a.at[:, cols], wt.at[0], wt_sem.at[0])
            la1 = _start_copy(pacc_a.at[:, cols], wt.at[1], wt_sem.at[1])
            lb0 = _start_copy(recv_b.at[:, cols], ot.at[0], ot_sem.at[0])
            lb1 = _start_copy(pacc_b.at[:, cols], ot.at[1], ot_sem.at[1])
            la0.wait()
            la1.wait()
            wt[1, :, :] = wt[0, :, :] + wt[1, :, :]
            am = jnp.maximum(am, jnp.max(jnp.abs(wt[1, :, :])))
            sa = _start_copy(wt.at[1], acc_a.at[:, cols], st_sem.at[0])
            lb0.wait()
            lb1.wait()
            ot[1, :, :] = ot[0, :, :] + ot[1, :, :]
            am = jnp.maximum(am, jnp.max(jnp.abs(ot[1, :, :])))
            sb = _start_copy(ot.at[1], acc_b.at[:, cols], st_sem.at[1])
            sa.wait()
            sb.wait()
        return am

    gemm_half(my, 0, acc_a, 0)
    gemm_half(my, NH, acc_b, 1)

    barrier = pltpu.get_barrier_semaphore()
    for nbr in (left, right):
        pl.semaphore_signal(barrier, inc=1, device_id=(nbr,),
                            device_id_type=pl.DeviceIdType.MESH)
    pl.semaphore_wait(barrier, 2)

    def rs_body(s, amax):
        cw = pltpu.make_async_remote_copy(
            src_ref=acc_a, dst_ref=recv_a,
            send_sem=send_a, recv_sem=recv_a_sem,
            device_id=(right,), device_id_type=pl.DeviceIdType.MESH)
        ccw = pltpu.make_async_remote_copy(
            src_ref=acc_b, dst_ref=recv_b,
            send_sem=send_b, recv_sem=recv_b_sem,
            device_id=(left,), device_id_type=pl.DeviceIdType.MESH)
        cw.start()
        ccw.start()
        gemm_half((my - s - 1) % N_DEV, 0, pacc_a, 0)
        gemm_half((my + s + 1) % N_DEV, NH, pacc_b, 1)
        cw.wait()
        ccw.wait()
        am = add_pass()
        nbarrier()
        return jnp.where(s == N_DEV - 2, am, amax)

    amax = lax.fori_loop(0, N_DEV - 1, rs_body, jnp.float32(0.0))

    amax_all[pl.ds(my, 1), :] = jnp.full((1, 128), amax, jnp.float32)
    for d in range(1, N_DEV):
        rd = pltpu.make_async_remote_copy(
            src_ref=amax_all.at[pl.ds(my, 1), :],
            dst_ref=amax_all.at[pl.ds(my, 1), :],
            send_sem=amax_send.at[d], recv_sem=amax_recv.at[d],
            device_id=((my + d) % N_DEV,),
            device_id_type=pl.DeviceIdType.MESH)
        rd.start()
    for d in range(1, N_DEV):
        wr = pltpu.make_async_remote_copy(
            src_ref=amax_all.at[pl.ds(my, 1), :],
            dst_ref=amax_all.at[pl.ds((my - d) % N_DEV, 1), :],
            send_sem=amax_send.at[d], recv_sem=amax_recv.at[d],
            device_id=(left,), device_id_type=pl.DeviceIdType.MESH)
        wr.wait_recv()
        wr.wait_send()

    gmax = jnp.max(amax_all[...])
    inv_scale = 127.0 / gmax
    scale = gmax / 127.0

    own_a = (my + 1) % N_DEV
    own_b = (my - 1) % N_DEV
    for srcbuf, qdst, c, off in ((acc_a, qa, own_a, 0),
                                 (acc_b, qb, own_b, NH)):
        for n in range(NTH):
            cols = pl.ds(n * BN, BN)
            _start_copy(srcbuf.at[:, cols], wt.at[0], wt_sem.at[0]).wait()
            qdst[0, :, cols] = jnp.clip(jnp.round(wt[0, :, :] * inv_scale),
                                        -127.0, 127.0).astype(jnp.int8)
            ot[0, :, :] = qdst[0, :, cols].astype(jnp.float32) * scale
            _start_copy(ot.at[0],
                        out_ref.at[pl.ds(c * CHUNK, CHUNK),
                                   pl.ds(off + n * BN, BN)],
                        ot_sem.at[0]).wait()

    def dequant_half(qbuf, slot, c, off):
        for n in range(NTH):
            cols = pl.ds(n * BN, BN)
            ot[0, :, :] = qbuf[slot, :, cols].astype(jnp.float32) * scale
            _start_copy(ot.at[0],
                        out_ref.at[pl.ds(c * CHUNK, CHUNK),
                                   pl.ds(off + n * BN, BN)],
                        ot_sem.at[0]).wait()

    for s in range(N_DEV - 1):
        cw = pltpu.make_async_remote_copy(
            src_ref=qa.at[s % 2], dst_ref=qa.at[(s + 1) % 2],
            send_sem=send_a, recv_sem=recv_a_sem,
            device_id=(right,), device_id_type=pl.DeviceIdType.MESH)
        ccw = pltpu.make_async_remote_copy(
            src_ref=qb.at[s % 2], dst_ref=qb.at[(s + 1) % 2],
            send_sem=send_b, recv_sem=recv_b_sem,
            device_id=(left,), device_id_type=pl.DeviceIdType.MESH)
        cw.start()
        ccw.start()
        if s > 0:
            dequant_half(qa, s % 2, (my - s + 1) % N_DEV, 0)
            dequant_half(qb, s % 2, (my + s - 1) % N_DEV, NH)
        cw.wait()
        ccw.wait()
        nbarrier()
    dequant_half(qa, (N_DEV - 1) % 2, (my - N_DEV + 2) % N_DEV, 0)
    dequant_half(qb, (N_DEV - 1) % 2, (my + N_DEV - 2) % N_DEV, NH)


def kernel(x, w_mat):
    outs = pl.pallas_call(
        _ar_body,
        in_specs=[pl.BlockSpec(memory_space=pl.ANY)] * 2,
        out_specs=[pl.BlockSpec(memory_space=pl.ANY)] * 7,
        out_shape=[
            jax.ShapeDtypeStruct((M, N), jnp.float32),
            jax.ShapeDtypeStruct((CHUNK, NH), jnp.float32),
            jax.ShapeDtypeStruct((CHUNK, NH), jnp.float32),
            jax.ShapeDtypeStruct((CHUNK, NH), jnp.float32),
            jax.ShapeDtypeStruct((CHUNK, NH), jnp.float32),
            jax.ShapeDtypeStruct((CHUNK, NH), jnp.float32),
            jax.ShapeDtypeStruct((CHUNK, NH), jnp.float32),
        ],
        scratch_shapes=[
            pltpu.VMEM((2, CHUNK, K_SHARD), jnp.float32),
            pltpu.VMEM((2, K_SHARD, BN), jnp.float32),
            pltpu.VMEM((2, CHUNK, BN), jnp.float32),
            pltpu.VMEM((2, CHUNK, NH), jnp.int8),
            pltpu.VMEM((2, CHUNK, NH), jnp.int8),
            pltpu.VMEM((N_DEV, 128), jnp.float32),
            pltpu.SemaphoreType.DMA((2,)),
            pltpu.SemaphoreType.DMA((2,)),
            pltpu.SemaphoreType.DMA((2,)),
            pltpu.SemaphoreType.DMA((2,)),
            pltpu.SemaphoreType.DMA,
            pltpu.SemaphoreType.DMA,
            pltpu.SemaphoreType.DMA,
            pltpu.SemaphoreType.DMA,
            pltpu.SemaphoreType.DMA((N_DEV,)),
            pltpu.SemaphoreType.DMA((N_DEV,)),
            pltpu.SemaphoreType.REGULAR,
        ],
        compiler_params=pltpu.CompilerParams(
            collective_id=0,
            vmem_limit_bytes=62 * 1024 * 1024,
        ),
    )(x, w_mat)
    return outs[0]


# device time: 711937 ns/iter; 2.4738x vs baseline; 1.3734x over previous
import jax
import jax.numpy as jnp
from jax import lax
from jax.experimental import pallas as pl
from jax.experimental.pallas import tpu as pltpu

N_DEV = 4
M, K_SHARD, N = 4096, 1024, 8192
CHUNK = M // N_DEV
NH = N // 2
BN = 2048
NTH = NH // BN


def _start_copy(src, dst, sem):
    cp = pltpu.make_async_copy(src, dst, sem)
    cp.start()
    return cp


def _ar_body(x_ref, w_ref, out_ref,
             acc_a, acc_b, pacc_a, pacc_b, recv_a, recv_b,
             xc, wt, ot, otb, qa, qb, amax_all,
             xc_sem, wt_sem, ot_sem, st_sem,
             send_a, recv_a_sem, send_b, recv_b_sem,
             amax_send, amax_recv, credit_sem):
    my = lax.axis_index("i")
    left = (my + N_DEV - 1) % N_DEV
    right = (my + 1) % N_DEV

    def nbarrier():
        for nbr in (left, right):
            pl.semaphore_signal(credit_sem, inc=1, device_id=(nbr,),
                                device_id_type=pl.DeviceIdType.MESH)
        pl.semaphore_wait(credit_sem, 2)

    def gemm_half(c, off, dst, to_bf16):
        cpx = _start_copy(x_ref.at[pl.ds(c * CHUNK, CHUNK), :], xc, xc_sem)
        w0 = _start_copy(w_ref.at[:, pl.ds(off, BN)], wt.at[0],
                         wt_sem.at[0])
        w1 = _start_copy(w_ref.at[:, pl.ds(off + BN, BN)], wt.at[1],
                         wt_sem.at[1])
        cpx.wait()
        w0.wait()
        ot[0, :, :] = jnp.dot(xc[...], wt[0, :, :],
                              preferred_element_type=jnp.float32)
        if to_bf16:
            otb[0, :, :] = ot[0, :, :].astype(jnp.bfloat16)
            s0 = _start_copy(otb.at[0], dst.at[:, pl.ds(0, BN)],
                             ot_sem.at[0])
        else:
            s0 = _start_copy(ot.at[0], dst.at[:, pl.ds(0, BN)],
                             ot_sem.at[0])
        w1.wait()
        ot[1, :, :] = jnp.dot(xc[...], wt[1, :, :],
                              preferred_element_type=jnp.float32)
        if to_bf16:
            otb[1, :, :] = ot[1, :, :].astype(jnp.bfloat16)
            s1 = _start_copy(otb.at[1], dst.at[:, pl.ds(BN, BN)],
                             ot_sem.at[1])
        else:
            s1 = _start_copy(ot.at[1], dst.at[:, pl.ds(BN, BN)],
                             ot_sem.at[1])
        s0.wait()
        s1.wait()

    def add_pass():
        am = jnp.float32(0.0)
        for n in range(NTH):
            cols = pl.ds(n * BN, BN)
            la0 = _start_copy(recv_a.at[:, cols], otb.at[0], ot_sem.at[0])
            la1 = _start_copy(pacc_a.at[:, cols], wt.at[0], wt_sem.at[0])
            lb0 = _start_copy(recv_b.at[:, cols], otb.at[1], ot_sem.at[1])
            lb1 = _start_copy(pacc_b.at[:, cols], wt.at[1], wt_sem.at[1])
            la0.wait()
            la1.wait()
            wt[0, :, :] = otb[0, :, :].astype(jnp.float32) + wt[0, :, :]
            am = jnp.maximum(am, jnp.max(jnp.abs(wt[0, :, :])))
            otb[0, :, :] = wt[0, :, :].astype(jnp.bfloat16)
            sa = _start_copy(otb.at[0], acc_a.at[:, cols], st_sem.at[0])
            fa = _start_copy(wt.at[0], pacc_a.at[:, cols], wt_sem.at[0])
            lb0.wait()
            lb1.wait()
            wt[1, :, :] = otb[1, :, :].astype(jnp.float32) + wt[1, :, :]
            am = jnp.maximum(am, jnp.max(jnp.abs(wt[1, :, :])))
            otb[1, :, :] = wt[1, :, :].astype(jnp.bfloat16)
            sb = _start_copy(otb.at[1], acc_b.at[:, cols], st_sem.at[1])
            fb = _start_copy(wt.at[1], pacc_b.at[:, cols], wt_sem.at[1])
            sa.wait()
            fa.wait()
            sb.wait()
            fb.wait()
        return am

    gemm_half(my, 0, acc_a, True)
    gemm_half(my, NH, acc_b, True)

    barrier = pltpu.get_barrier_semaphore()
    for nbr in (left, right):
        pl.semaphore_signal(barrier, inc=1, device_id=(nbr,),
                            device_id_type=pl.DeviceIdType.MESH)
    pl.semaphore_wait(barrier, 2)

    def rs_body(s, amax):
        cw = pltpu.make_async_remote_copy(
            src_ref=acc_a, dst_ref=recv_a,
            send_sem=send_a, recv_sem=recv_a_sem,
            device_id=(right,), device_id_type=pl.DeviceIdType.MESH)
        ccw = pltpu.make_async_remote_copy(
            src_ref=acc_b, dst_ref=recv_b,
            send_sem=send_b, recv_sem=recv_b_sem,
            device_id=(left,), device_id_type=pl.DeviceIdType.MESH)
        cw.start()
        ccw.start()
        gemm_half((my - s - 1) % N_DEV, 0, pacc_a, False)
        gemm_half((my + s + 1) % N_DEV, NH, pacc_b, False)
        cw.wait()
        ccw.wait()
        am = add_pass()
        nbarrier()
        return jnp.where(s == N_DEV - 2, am, amax)

    amax = lax.fori_loop(0, N_DEV - 1, rs_body, jnp.float32(0.0))

    amax_all[pl.ds(my, 1), :] = jnp.full((1, 128), amax, jnp.float32)
    for d in range(1, N_DEV):
        rd = pltpu.make_async_remote_copy(
            src_ref=amax_all.at[pl.ds(my, 1), :],
            dst_ref=amax_all.at[pl.ds(my, 1), :],
            send_sem=amax_send.at[d], recv_sem=amax_recv.at[d],
            device_id=((my + d) % N_DEV,),
            device_id_type=pl.DeviceIdType.MESH)
        rd.start()
    for d in range(1, N_DEV):
        wr = pltpu.make_async_remote_copy(
            src_ref=amax_all.at[pl.ds(my, 1), :],
            dst_ref=amax_all.at[pl.ds((my - d) % N_DEV, 1), :],
            send_sem=amax_send.at[d], recv_sem=amax_recv.at[d],
            device_id=(left,), device_id_type=pl.DeviceIdType.MESH)
        wr.wait_recv()
        wr.wait_send()

    gmax = jnp.max(amax_all[...])
    inv_scale = 127.0 / gmax
    scale = gmax / 127.0

    own_a = (my + 1) % N_DEV
    own_b = (my - 1) % N_DEV
    for srcbuf, qdst, c, off in ((pacc_a, qa, own_a, 0),
                                 (pacc_b, qb, own_b, NH)):
        for n in range(NTH):
            cols = pl.ds(n * BN, BN)
            _start_copy(srcbuf.at[:, cols], wt.at[0], wt_sem.at[0]).wait()
            qdst[0, :, cols] = jnp.clip(jnp.round(wt[0, :, :] * inv_scale),
                                        -127.0, 127.0).astype(jnp.int8)
            ot[0, :, :] = qdst[0, :, cols].astype(jnp.float32) * scale
            _start_copy(ot.at[0],
                        out_ref.at[pl.ds(c * CHUNK, CHUNK),
                                   pl.ds(off + n * BN, BN)],
                        ot_sem.at[0]).wait()

    def dequant_half(qbuf, slot, c, off):
        for n in range(NTH):
            cols = pl.ds(n * BN, BN)
            ot[0, :, :] = qbuf[slot, :, cols].astype(jnp.float32) * scale
            _start_copy(ot.at[0],
                        out_ref.at[pl.ds(c * CHUNK, CHUNK),
                                   pl.ds(off + n * BN, BN)],
                        ot_sem.at[0]).wait()

    for s in range(N_DEV - 1):
        cw = pltpu.make_async_remote_copy(
            src_ref=qa.at[s % 2], dst_ref=qa.at[(s + 1) % 2],
            send_sem=send_a, recv_sem=recv_a_sem,
            device_id=(right,), device_id_type=pl.DeviceIdType.MESH)
        ccw = pltpu.make_async_remote_copy(
            src_ref=qb.at[s % 2], dst_ref=qb.at[(s + 1) % 2],
            send_sem=send_b, recv_sem=recv_b_sem,
            device_id=(left,), device_id_type=pl.DeviceIdType.MESH)
        cw.start()
        ccw.start()
        if s > 0:
            dequant_half(qa, s % 2, (my - s + 1) % N_DEV, 0)
            dequant_half(qb, s % 2, (my + s - 1) % N_DEV, NH)
        cw.wait()
        ccw.wait()
        nbarrier()
    dequant_half(qa, (N_DEV - 1) % 2, (my - N_DEV + 2) % N_DEV, 0)
    dequant_half(qb, (N_DEV - 1) % 2, (my + N_DEV - 2) % N_DEV, NH)


def kernel(x, w_mat):
    outs = pl.pallas_call(
        _ar_body,
        in_specs=[pl.BlockSpec(memory_space=pl.ANY)] * 2,
        out_specs=[pl.BlockSpec(memory_space=pl.ANY)] * 7,
        out_shape=[
            jax.ShapeDtypeStruct((M, N), jnp.float32),
            jax.ShapeDtypeStruct((CHUNK, NH), jnp.bfloat16),
            jax.ShapeDtypeStruct((CHUNK, NH), jnp.bfloat16),
            jax.ShapeDtypeStruct((CHUNK, NH), jnp.float32),
            jax.ShapeDtypeStruct((CHUNK, NH), jnp.float32),
            jax.ShapeDtypeStruct((CHUNK, NH), jnp.bfloat16),
            jax.ShapeDtypeStruct((CHUNK, NH), jnp.bfloat16),
        ],
        scratch_shapes=[
            pltpu.VMEM((CHUNK, K_SHARD), jnp.float32),
            pltpu.VMEM((2, K_SHARD, BN), jnp.float32),
            pltpu.VMEM((2, CHUNK, BN), jnp.float32),
            pltpu.VMEM((2, CHUNK, BN), jnp.bfloat16),
            pltpu.VMEM((2, CHUNK, NH), jnp.int8),
            pltpu.VMEM((2, CHUNK, NH), jnp.int8),
            pltpu.VMEM((N_DEV, 128), jnp.float32),
            pltpu.SemaphoreType.DMA,
            pltpu.SemaphoreType.DMA((2,)),
            pltpu.SemaphoreType.DMA((2,)),
            pltpu.SemaphoreType.DMA((2,)),
            pltpu.SemaphoreType.DMA,
            pltpu.SemaphoreType.DMA,
            pltpu.SemaphoreType.DMA,
            pltpu.SemaphoreType.DMA,
            pltpu.SemaphoreType.DMA((N_DEV,)),
            pltpu.SemaphoreType.DMA((N_DEV,)),
            pltpu.SemaphoreType.REGULAR,
        ],
        compiler_params=pltpu.CompilerParams(
            collective_id=0,
            vmem_limit_bytes=67_000_000,
        ),
    )(x, w_mat)
    return outs[0]
